# baseline (speedup 1.0000x reference)
import functools
import jax
import jax.numpy as jnp
from jax import lax
from jax.experimental import pallas as pl
from jax.experimental.pallas import tpu as pltpu
from jax.experimental.pallas import tpu_sc as plsc

B, S, D = 1024, 200, 128
NC, NS = 2, 16
NW = NC * NS
BW = B // NW
ROWS_W = BW * S
CH = 400
NCH = ROWS_W // CH


def _sc_body(in_hbm, pos_hbm, mask_hbm, out_hbm,
             buf0, buf1, pos_v, idx_v, mask_v, rows_v,
             in_sem0, in_sem1, out_sem0, out_sem1, sc_sem):
    wid = lax.axis_index("s") * NC + lax.axis_index("c")
    row_base = wid * ROWS_W
    bufs = (buf0, buf1)
    in_sems = (in_sem0, in_sem1)
    out_sems = (out_sem0, out_sem1)

    h_in = [None] * NCH
    h_out = [None] * NCH
    for g in range(NCH):
        if g >= 2:
            h_out[g - 2].wait()
        r0 = row_base + g * CH
        h_in[g] = pltpu.async_copy(in_hbm.at[pl.ds(r0, CH)], bufs[g % 2], in_sems[g % 2])
        if g >= 1:
            h_in[g - 1].wait()
            r0p = row_base + (g - 1) * CH
            h_out[g - 1] = pltpu.async_copy(bufs[(g - 1) % 2], out_hbm.at[pl.ds(r0p, CH)],
                                            out_sems[(g - 1) % 2])
    h_in[NCH - 1].wait()
    r0p = row_base + (NCH - 1) * CH
    h_out[NCH - 1] = pltpu.async_copy(bufs[(NCH - 1) % 2], out_hbm.at[pl.ds(r0p, CH)],
                                      out_sems[(NCH - 1) % 2])
    h_out[NCH - 2].wait()
    h_out[NCH - 1].wait()

    pltpu.sync_copy(pos_hbm.at[pl.ds(wid * BW, BW)], pos_v)
    for j in range(BW // 16):
        batch = wid * BW + j * 16 + lax.iota(jnp.int32, 16)
        idx_v[pl.ds(j * 16, 16)] = pos_v[pl.ds(j * 16, 16)] + batch * S
    pltpu.sync_copy(mask_hbm, mask_v)
    chunks = [mask_v[0, pl.ds(c * 16, 16)] for c in range(D // 16)]
    for r in range(BW):
        for c in range(D // 16):
            rows_v[r, pl.ds(c * 16, 16)] = chunks[c]
    pltpu.async_copy(rows_v, out_hbm.at[idx_v], sc_sem).wait()


_sc_call = functools.partial(
    pl.kernel,
    out_type=jax.ShapeDtypeStruct((B * S, D), jnp.float32),
    mesh=plsc.VectorSubcoreMesh(core_axis_name="c", subcore_axis_name="s"),
    scratch_types=[
        pltpu.VMEM((CH, D), jnp.float32),
        pltpu.VMEM((CH, D), jnp.float32),
        pltpu.VMEM((BW,), jnp.int32),
        pltpu.VMEM((BW,), jnp.int32),
        pltpu.VMEM((1, D), jnp.float32),
        pltpu.VMEM((BW, D), jnp.float32),
        pltpu.SemaphoreType.DMA,
        pltpu.SemaphoreType.DMA,
        pltpu.SemaphoreType.DMA,
        pltpu.SemaphoreType.DMA,
        pltpu.SemaphoreType.DMA,
    ],
)(_sc_body)


def kernel(inputs, categories, mask_positions, tokens_embedding):
    del categories
    pos = mask_positions.reshape(B).astype(jnp.int32)
    out = _sc_call(inputs.reshape(B * S, D), pos, tokens_embedding)
    return out.reshape(B, S, D)
